# baseline (speedup 1.0000x reference)
import functools

import jax
import jax.numpy as jnp
from jax import lax
from jax.experimental import pallas as pl
from jax.experimental.pallas import tpu as pltpu
from jax.experimental.pallas import tpu_sc as plsc

VOCAB = 150528
HIDDEN = 12288
TOKENS = 8192

NC, NS = 2, 16
NW = NC * NS

TW = 16
SW = NW - TW
RT = 160
RS = (TOKENS - TW * RT) // SW

K = 4
NBUF = 2
CHT = RT // K
GT = CHT // NBUF

NGS = RS // 16
SBUF = 4

_mesh = plsc.VectorSubcoreMesh(
    core_axis_name="c", subcore_axis_name="s", num_cores=NC, num_subcores=NS
)


@functools.partial(
    pl.kernel,
    mesh=_mesh,
    out_type=jax.ShapeDtypeStruct((TOKENS, HIDDEN), jnp.float32),
    scratch_types=[
        pltpu.VMEM((CHT, K), jnp.int32),
        pltpu.VMEM((NGS, 16), jnp.int32),
        [pltpu.VMEM((K, HIDDEN), jnp.float32) for _ in range(NBUF)],
        pltpu.VMEM_SHARED((NS // 2, SBUF, HIDDEN), jnp.float32),
        [pltpu.SemaphoreType.DMA for _ in range(NBUF)],
        [pltpu.SemaphoreType.DMA for _ in range(NBUF)],
        [pltpu.SemaphoreType.DMA for _ in range(SBUF)],
        [pltpu.SemaphoreType.DMA for _ in range(SBUF)],
    ],
)
def _sc_gather(
    idxt_hbm, idxs_hbm, table_hbm, out_hbm,
    idxt_v, idxs_v, bufs, spbuf, gsem, wsem, sgsem, swsem,
):
    cid = lax.axis_index("c")
    sid = lax.axis_index("s")
    wid = sid * NC + cid

    @pl.when(wid < TW)
    def _tile_path():
        base = wid * RT
        pltpu.sync_copy(idxt_hbm.at[wid], idxt_v)

        def gather_desc(c, b):
            return pltpu.make_async_copy(
                table_hbm.at[idxt_v.at[c]], bufs[b], gsem[b]
            )

        def write_desc(c, b):
            return pltpu.make_async_copy(
                bufs[b], out_hbm.at[pl.ds(base + c * K, K)], wsem[b]
            )

        for b in range(NBUF):
            gather_desc(b, b).start()

        def body(g, carry):
            for b in range(NBUF):
                c = NBUF * g + b
                gather_desc(c, b).wait()
                write_desc(c, b).start()
            for b in range(NBUF):
                c = NBUF * g + b
                write_desc(c, b).wait()
                gather_desc(c + NBUF, b).start()
            return carry

        lax.fori_loop(0, GT - 1, body, 0)

        for b in range(NBUF):
            c = CHT - NBUF + b
            gather_desc(c, b).wait()
            write_desc(c, b).start()
        for b in range(NBUF):
            write_desc(CHT - NBUF + b, b).wait()

    @pl.when(wid >= TW)
    def _spmem_path():
        swid = wid - TW
        base = TW * RT + swid * RS
        pltpu.sync_copy(idxs_hbm.at[swid], idxs_v)

        def sgather_desc(row, b):
            return pltpu.make_async_copy(
                table_hbm.at[row], spbuf.at[sid - NS // 2, b], sgsem[b]
            )

        def swrite_desc(r, b):
            return pltpu.make_async_copy(
                spbuf.at[sid - NS // 2, b], out_hbm.at[base + r], swsem[b]
            )

        def body(g, carry):
            v = idxs_v.at[g][...]
            for h in range(16 // SBUF):
                for j in range(SBUF):
                    sgather_desc(v[SBUF * h + j], j).start()
                for j in range(SBUF):
                    sgather_desc(v[SBUF * h + j], j).wait()
                for j in range(SBUF):
                    swrite_desc(16 * g + SBUF * h + j, j).start()
                for j in range(SBUF):
                    swrite_desc(16 * g + SBUF * h + j, j).wait()
            return carry

        lax.fori_loop(0, NGS, body, 0)


def kernel(input_ids, word_embeddings):
    b, s = input_ids.shape
    flat = input_ids.reshape(TOKENS)
    idx_t = flat[: TW * RT].reshape(TW, CHT, K)
    idx_s = flat[TW * RT :].reshape(SW, NGS, 16)
    out = _sc_gather(idx_t, idx_s, word_embeddings)
    return out.reshape(b, s, HIDDEN)


# speedup vs baseline: 1.1296x; 1.1296x over previous
import functools

import jax
import jax.numpy as jnp
from jax import lax
from jax.experimental import pallas as pl
from jax.experimental.pallas import tpu as pltpu
from jax.experimental.pallas import tpu_sc as plsc

VOCAB = 150528
HIDDEN = 12288
TOKENS = 8192

NC, NS = 2, 16
NW = NC * NS
ROWS_PER_W = TOKENS // NW
NG = ROWS_PER_W // 16
SBUF = 8

_mesh = plsc.VectorSubcoreMesh(
    core_axis_name="c", subcore_axis_name="s", num_cores=NC, num_subcores=NS
)


@functools.partial(
    pl.kernel,
    mesh=_mesh,
    out_type=jax.ShapeDtypeStruct((TOKENS, HIDDEN), jnp.float32),
    scratch_types=[
        pltpu.VMEM((NG, 16), jnp.int32),
        pltpu.VMEM_SHARED((NS, SBUF, HIDDEN), jnp.float32),
        [pltpu.SemaphoreType.DMA for _ in range(SBUF)],
        [pltpu.SemaphoreType.DMA for _ in range(SBUF)],
    ],
)
def _sc_gather(idx_hbm, table_hbm, out_hbm, idx_v, spbuf, gsem, wsem):
    cid = lax.axis_index("c")
    sid = lax.axis_index("s")
    wid = sid * NC + cid
    base = wid * ROWS_PER_W
    pltpu.sync_copy(idx_hbm.at[wid], idx_v)

    def gather_start(row, j):
        pltpu.make_async_copy(table_hbm.at[row], spbuf.at[sid, j], gsem[j]).start()

    def gather_wait(j):
        pltpu.make_async_copy(table_hbm.at[0], spbuf.at[sid, j], gsem[j]).wait()

    def write_start(r, j):
        pltpu.make_async_copy(
            spbuf.at[sid, j], out_hbm.at[base + r], wsem[j]
        ).start()

    def write_wait(r, j):
        pltpu.make_async_copy(
            spbuf.at[sid, j], out_hbm.at[base + r], wsem[j]
        ).wait()

    def chunk_gather(v, lane_cl, cl):
        b = cl % 2
        for j in range(4):
            gather_start(v[4 * lane_cl + j], 4 * b + j)

    v0 = idx_v.at[0][...]
    chunk_gather(v0, 0, 0)
    chunk_gather(v0, 1, 1)

    def body(g, carry):
        v = idx_v.at[g][...]
        vn = idx_v.at[g + 1][...]
        for pair in range(2):
            for b in range(2):
                cl = 2 * pair + b
                r0 = 16 * g + 4 * cl
                for j in range(4):
                    gather_wait(4 * b + j)
                for j in range(4):
                    write_start(r0 + j, 4 * b + j)
            for b in range(2):
                cl = 2 * pair + b
                r0 = 16 * g + 4 * cl
                for j in range(4):
                    write_wait(r0 + j, 4 * b + j)
                if cl < 2:
                    chunk_gather(v, cl + 2, cl + 2)
                else:
                    chunk_gather(vn, cl - 2, cl - 2)
        return carry

    lax.fori_loop(0, NG - 1, body, 0)

    g = NG - 1
    vlast = idx_v.at[g][...]
    for pair in range(2):
        for b in range(2):
            cl = 2 * pair + b
            r0 = 16 * g + 4 * cl
            for j in range(4):
                gather_wait(4 * b + j)
            for j in range(4):
                write_start(r0 + j, 4 * b + j)
        for b in range(2):
            cl = 2 * pair + b
            r0 = 16 * g + 4 * cl
            for j in range(4):
                write_wait(r0 + j, 4 * b + j)
            if pair == 0:
                chunk_gather(vlast, cl + 2, cl + 2)


def kernel(input_ids, word_embeddings):
    b, s = input_ids.shape
    idx = input_ids.reshape(NW, NG, 16)
    out = _sc_gather(idx, word_embeddings)
    return out.reshape(b, s, HIDDEN)


# speedup vs baseline: 1.1304x; 1.0007x over previous
import functools

import jax
import jax.numpy as jnp
from jax import lax
from jax.experimental import pallas as pl
from jax.experimental.pallas import tpu as pltpu
from jax.experimental.pallas import tpu_sc as plsc

VOCAB = 150528
HIDDEN = 12288
TOKENS = 8192

NC, NS = 2, 16
NW = NC * NS
ROWS_PER_W = TOKENS // NW
NG = ROWS_PER_W // 16

_mesh = plsc.VectorSubcoreMesh(
    core_axis_name="c", subcore_axis_name="s", num_cores=NC, num_subcores=NS
)


@functools.partial(
    pl.kernel,
    mesh=_mesh,
    out_type=jax.ShapeDtypeStruct((TOKENS, HIDDEN), jnp.float32),
    scratch_types=[
        pltpu.VMEM((NG, 16), jnp.int32),
        pltpu.VMEM_SHARED((NS, 2, 4, HIDDEN), jnp.float32),
        [pltpu.SemaphoreType.DMA for _ in range(8)],
        [pltpu.SemaphoreType.DMA for _ in range(2)],
    ],
)
def _sc_gather(idx_hbm, table_hbm, out_hbm, idx_v, spbuf, gsem, wsem):
    cid = lax.axis_index("c")
    sid = lax.axis_index("s")
    wid = sid * NC + cid
    base = wid * ROWS_PER_W
    pltpu.sync_copy(idx_hbm.at[wid], idx_v)

    def gather_start(row, b, j):
        pltpu.make_async_copy(
            table_hbm.at[row], spbuf.at[sid, b, j], gsem[4 * b + j]
        ).start()

    def gather_wait(b, j):
        pltpu.make_async_copy(
            table_hbm.at[0], spbuf.at[sid, b, j], gsem[4 * b + j]
        ).wait()

    def write_desc(r0, b):
        return pltpu.make_async_copy(
            spbuf.at[sid, b], out_hbm.at[pl.ds(base + r0, 4)], wsem[b]
        )

    def chunk_gather(v, lane_cl, cl):
        b = cl % 2
        for j in range(4):
            gather_start(v[4 * lane_cl + j], b, j)

    v0 = idx_v.at[0][...]
    chunk_gather(v0, 0, 0)
    chunk_gather(v0, 1, 1)

    def body(g, carry):
        v = idx_v.at[g][...]
        vn = idx_v.at[g + 1][...]
        for pair in range(2):
            for b in range(2):
                cl = 2 * pair + b
                r0 = 16 * g + 4 * cl
                for j in range(4):
                    gather_wait(b, j)
                write_desc(r0, b).start()
            for b in range(2):
                cl = 2 * pair + b
                r0 = 16 * g + 4 * cl
                write_desc(r0, b).wait()
                if cl < 2:
                    chunk_gather(v, cl + 2, cl + 2)
                else:
                    chunk_gather(vn, cl - 2, cl - 2)
        return carry

    lax.fori_loop(0, NG - 1, body, 0)

    g = NG - 1
    vlast = idx_v.at[g][...]
    for pair in range(2):
        for b in range(2):
            cl = 2 * pair + b
            r0 = 16 * g + 4 * cl
            for j in range(4):
                gather_wait(b, j)
            write_desc(r0, b).start()
        for b in range(2):
            cl = 2 * pair + b
            r0 = 16 * g + 4 * cl
            write_desc(r0, b).wait()
            if pair == 0:
                chunk_gather(vlast, cl + 2, cl + 2)


def kernel(input_ids, word_embeddings):
    b, s = input_ids.shape
    idx = input_ids.reshape(NW, NG, 16)
    out = _sc_gather(idx, word_embeddings)
    return out.reshape(b, s, HIDDEN)


# speedup vs baseline: 1.1307x; 1.0003x over previous
import functools

import jax
import jax.numpy as jnp
from jax import lax
from jax.experimental import pallas as pl
from jax.experimental.pallas import tpu as pltpu
from jax.experimental.pallas import tpu_sc as plsc

VOCAB = 150528
HIDDEN = 12288
TOKENS = 8192

NC, NS = 2, 16
NW = NC * NS
ROWS_PER_W = TOKENS // NW
NG = ROWS_PER_W // 16

_mesh = plsc.VectorSubcoreMesh(
    core_axis_name="c", subcore_axis_name="s", num_cores=NC, num_subcores=NS
)


@functools.partial(
    pl.kernel,
    mesh=_mesh,
    out_type=jax.ShapeDtypeStruct((TOKENS, HIDDEN), jnp.float32),
    scratch_types=[
        pltpu.VMEM((NG, 16), jnp.int32),
        pltpu.VMEM_SHARED((NS, 2, 4, HIDDEN), jnp.float32),
        [pltpu.SemaphoreType.DMA for _ in range(2)],
        [pltpu.SemaphoreType.DMA for _ in range(2)],
    ],
)
def _sc_gather(idx_hbm, table_hbm, out_hbm, idx_v, spbuf, gsem, wsem):
    cid = lax.axis_index("c")
    sid = lax.axis_index("s")
    wid = sid * NC + cid
    base = wid * ROWS_PER_W
    pltpu.sync_copy(idx_hbm.at[wid], idx_v)

    def gather_start(row, b, j):
        pltpu.make_async_copy(
            table_hbm.at[row], spbuf.at[sid, b, j], gsem[b]
        ).start()

    def gather_wait_set(b):
        pltpu.make_async_copy(
            table_hbm.at[pl.ds(0, 4)], spbuf.at[sid, b], gsem[b]
        ).wait()

    def write_desc(r0, b):
        return pltpu.make_async_copy(
            spbuf.at[sid, b], out_hbm.at[pl.ds(base + r0, 4)], wsem[b]
        )

    def chunk_gather(v, lane_cl, cl):
        b = cl % 2
        for j in range(4):
            gather_start(v[4 * lane_cl + j], b, j)

    v0 = idx_v.at[0][...]
    chunk_gather(v0, 0, 0)
    chunk_gather(v0, 1, 1)

    def body(g, carry):
        v = idx_v.at[g][...]
        vn = idx_v.at[g + 1][...]
        for pair in range(2):
            for b in range(2):
                cl = 2 * pair + b
                r0 = 16 * g + 4 * cl
                gather_wait_set(b)
                write_desc(r0, b).start()
            for b in range(2):
                cl = 2 * pair + b
                r0 = 16 * g + 4 * cl
                write_desc(r0, b).wait()
                if cl < 2:
                    chunk_gather(v, cl + 2, cl + 2)
                else:
                    chunk_gather(vn, cl - 2, cl - 2)
        return carry

    lax.fori_loop(0, NG - 1, body, 0)

    g = NG - 1
    vlast = idx_v.at[g][...]
    for pair in range(2):
        for b in range(2):
            cl = 2 * pair + b
            r0 = 16 * g + 4 * cl
            gather_wait_set(b)
            write_desc(r0, b).start()
        for b in range(2):
            cl = 2 * pair + b
            r0 = 16 * g + 4 * cl
            write_desc(r0, b).wait()
            if pair == 0:
                chunk_gather(vlast, cl + 2, cl + 2)


def kernel(input_ids, word_embeddings):
    b, s = input_ids.shape
    idx = input_ids.reshape(NW, NG, 16)
    out = _sc_gather(idx, word_embeddings)
    return out.reshape(b, s, HIDDEN)
